# Initial kernel scaffold; baseline (speedup 1.0000x reference)
import jax
import jax.numpy as jnp
from jax.experimental import pallas as pl


def kernel(e_s, pair, d_prob, hop_attn_logits, q_mask_logits):
    raise NotImplementedError("write your pallas kernel here")


# speedup vs baseline: 33.9789x; 33.9789x over previous
import functools

import jax
import jax.numpy as jnp
from jax import lax
from jax.experimental import pallas as pl
from jax.experimental.pallas import tpu as pltpu
from jax.experimental.pallas import tpu_sc as plsc

NC = 2
NS = 16
NW = NC * NS
LANES = 16
BSZ = 4

CHUNK = 2000


@functools.partial(jax.jit, static_argnums=(5, 6))
def _step_call(tab_in, sub, obj, p, zeros, nent_pad, ept):
    e_pad = sub.shape[0]
    nchunk = ept // CHUNK
    rpt = nent_pad // NS

    mesh = plsc.VectorSubcoreMesh(
        core_axis_name="c", subcore_axis_name="s",
        num_cores=NC, num_subcores=NS)

    @functools.partial(
        pl.kernel,
        out_type=jax.ShapeDtypeStruct((NC * BSZ * nent_pad,), jnp.float32),
        mesh=mesh,
        scratch_types=(
            [pltpu.VMEM_SHARED((nent_pad,), jnp.float32) for _ in range(BSZ)]
            + [pltpu.VMEM_SHARED((nent_pad,), jnp.float32) for _ in range(BSZ)]
            + [pltpu.VMEM((CHUNK,), jnp.int32) for _ in range(2)]
            + [pltpu.VMEM((CHUNK,), jnp.float32) for _ in range(2 * BSZ)]
            + [pltpu.VMEM((rpt,), jnp.float32)]
        ),
    )
    def step(tab_hbm, sub_hbm, obj_hbm, p_hbm, zeros_hbm, out_hbm,
             t0, t1, t2, t3, a0, a1, a2, a3,
             subc, objc, pc0, pc1, pc2, pc3, g0, g1, g2, g3, stage):
        tabs = (t0, t1, t2, t3)
        accs = (a0, a1, a2, a3)
        pcs = (pc0, pc1, pc2, pc3)
        gs = (g0, g1, g2, g3)
        c = lax.axis_index("c")
        s = lax.axis_index("s")
        wid = c * NS + s

        ent0 = s * rpt
        pltpu.sync_copy(zeros_hbm.at[pl.ds(ent0, rpt)], stage)
        for b in range(BSZ):
            pltpu.sync_copy(stage, accs[b].at[pl.ds(ent0, rpt)])
        for b in range(BSZ):
            pltpu.sync_copy(tab_hbm.at[pl.ds(b * nent_pad + ent0, rpt)],
                            stage)
            pltpu.sync_copy(stage, tabs[b].at[pl.ds(ent0, rpt)])
        plsc.subcore_barrier()

        def chunk_body(i, carry):
            base = wid * ept + i * CHUNK
            pltpu.sync_copy(sub_hbm.at[pl.ds(base, CHUNK)], subc)
            pltpu.sync_copy(obj_hbm.at[pl.ds(base, CHUNK)], objc)
            for b in range(BSZ):
                pltpu.sync_copy(p_hbm.at[pl.ds(b * e_pad + base, CHUNK)],
                                pcs[b])
            for b in range(BSZ):
                pltpu.sync_copy(tabs[b].at[subc], gs[b])
            def mul_body(v, carry2):
                sl = pl.ds(v * LANES, LANES)
                for b in range(BSZ):
                    gs[b][sl] = gs[b][sl] * pcs[b][sl]
                return carry2
            lax.fori_loop(0, CHUNK // LANES, mul_body, 0)
            for b in range(BSZ):
                pltpu.sync_copy(gs[b], accs[b].at[objc], add=True)
            return carry
        lax.fori_loop(0, nchunk, chunk_body, 0)

        plsc.subcore_barrier()
        for b in range(BSZ):
            pltpu.sync_copy(accs[b].at[pl.ds(ent0, rpt)], stage)
            pltpu.sync_copy(
                stage,
                out_hbm.at[pl.ds((c * BSZ + b) * nent_pad + ent0, rpt)])

    return step(tab_in, sub, obj, p, zeros)


def kernel(e_s, pair, d_prob, hop_attn_logits, q_mask_logits):
    num_steps, bsz, E = d_prob.shape
    num_ent = e_s.shape[1]

    nent_pad = -(-num_ent // (NS * LANES)) * (NS * LANES)
    e_pad = -(-E // (NW * CHUNK)) * (NW * CHUNK)
    ept = e_pad // NW

    sub = pair[:, 0]
    obj = pair[:, 1]
    if e_pad != E:
        sub = jnp.pad(sub, (0, e_pad - E))
        obj = jnp.pad(obj, (0, e_pad - E))
        d_prob = jnp.pad(d_prob, ((0, 0), (0, 0), (0, e_pad - E)))

    zeros = jnp.zeros((nent_pad,), jnp.float32)
    tab = jnp.pad(e_s, ((0, 0), (0, nent_pad - num_ent)))

    ent_probs = []
    for t in range(num_steps):
        parts = _step_call(tab.reshape(-1), sub, obj,
                           d_prob[t].reshape(-1), zeros, nent_pad, ept)
        parts = parts.reshape(NC, bsz, nent_pad)
        new_e = parts[0] + parts[1]
        tab = new_e / jnp.maximum(new_e, 1.0)
        ent_probs.append(tab[:, :num_ent])

    hop_attn = jax.nn.softmax(hop_attn_logits, axis=1)
    last_e = sum(ent_probs[t] * hop_attn[:, t:t + 1] for t in range(num_steps))
    m = (jnp.argmax(hop_attn, axis=1) == 1).astype(jnp.float32)[:, None] * e_s
    last_e = (1.0 - m) * last_e
    last_e = last_e * jax.nn.sigmoid(q_mask_logits)
    return last_e


# speedup vs baseline: 41.7464x; 1.2286x over previous
import functools

import jax
import jax.numpy as jnp
from jax import lax
from jax.experimental import pallas as pl
from jax.experimental.pallas import tpu as pltpu
from jax.experimental.pallas import tpu_sc as plsc

NC = 2
NS = 16
NW = NC * NS
LANES = 16
BSZ = 4

CHUNK = 2000


@functools.partial(jax.jit, static_argnums=(5, 6))
def _step_call(tab_in, sub, obj, p, zeros, nent_pad, ept):
    e_pad = sub.shape[0]
    nchunk = ept // CHUNK
    rpt = nent_pad // NS

    mesh = plsc.VectorSubcoreMesh(
        core_axis_name="c", subcore_axis_name="s",
        num_cores=NC, num_subcores=NS)

    @functools.partial(
        pl.kernel,
        out_type=jax.ShapeDtypeStruct((NC * BSZ * nent_pad,), jnp.float32),
        mesh=mesh,
        scratch_types=(
            [pltpu.VMEM_SHARED((nent_pad,), jnp.float32) for _ in range(BSZ)]
            + [pltpu.VMEM_SHARED((nent_pad,), jnp.float32) for _ in range(BSZ)]
            + [pltpu.VMEM((CHUNK,), jnp.int32) for _ in range(4)]
            + [pltpu.VMEM((CHUNK,), jnp.float32) for _ in range(4 * BSZ)]
            + [pltpu.VMEM((rpt,), jnp.float32)]
            + [pltpu.SemaphoreType.DMA for _ in range(6)]
        ),
    )
    def step(tab_hbm, sub_hbm, obj_hbm, p_hbm, zeros_hbm, out_hbm,
             t0, t1, t2, t3, a0, a1, a2, a3,
             sub0, sub1, obj0, obj1,
             pc00, pc01, pc02, pc03, pc10, pc11, pc12, pc13,
             g00, g01, g02, g03, g10, g11, g12, g13, stage,
             sin0, sin1, sg, ss0, ss1, sem_pro):
        tabs = (t0, t1, t2, t3)
        accs = (a0, a1, a2, a3)
        subc = (sub0, sub1)
        objc = (obj0, obj1)
        pcs = ((pc00, pc01, pc02, pc03), (pc10, pc11, pc12, pc13))
        gs = ((g00, g01, g02, g03), (g10, g11, g12, g13))
        sem_in = (sin0, sin1)
        sem_s = (ss0, ss1)
        c = lax.axis_index("c")
        s = lax.axis_index("s")
        wid = c * NS + s

        def issue_inputs(j, slot):
            base = wid * ept + j * CHUNK
            pltpu.async_copy(sub_hbm.at[pl.ds(base, CHUNK)], subc[slot],
                             sem_in[slot])
            pltpu.async_copy(obj_hbm.at[pl.ds(base, CHUNK)], objc[slot],
                             sem_in[slot])
            for b in range(BSZ):
                pltpu.async_copy(p_hbm.at[pl.ds(b * e_pad + base, CHUNK)],
                                 pcs[slot][b], sem_in[slot])

        def wait_inputs(slot):
            pltpu.make_async_copy(sub_hbm.at[pl.ds(0, CHUNK)], subc[slot],
                                  sem_in[slot]).wait()
            pltpu.make_async_copy(obj_hbm.at[pl.ds(0, CHUNK)], objc[slot],
                                  sem_in[slot]).wait()
            for b in range(BSZ):
                pltpu.make_async_copy(p_hbm.at[pl.ds(0, CHUNK)],
                                      pcs[slot][b], sem_in[slot]).wait()

        def wait_scatters(slot):
            for b in range(BSZ):
                pltpu.make_async_copy(gs[slot][b], accs[b].at[objc[slot]],
                                      sem_s[slot]).wait()

        ent0 = s * rpt
        pltpu.sync_copy(zeros_hbm.at[pl.ds(ent0, rpt)], stage)
        for b in range(BSZ):
            pltpu.sync_copy(stage, accs[b].at[pl.ds(ent0, rpt)])
        for b in range(BSZ):
            pltpu.sync_copy(tab_hbm.at[pl.ds(b * nent_pad + ent0, rpt)],
                            stage)
            pltpu.sync_copy(stage, tabs[b].at[pl.ds(ent0, rpt)])
        plsc.subcore_barrier()

        issue_inputs(0, 0)
        def chunk_body(i, carry):
            slot = lax.rem(i, 2)
            other = 1 - slot

            def slot_body(sl, ot):
                @pl.when(i > 0)
                def _():
                    wait_scatters(ot)
                nxt = jnp.minimum(i + 1, nchunk - 1)
                issue_inputs(nxt, ot)
                wait_inputs(sl)
                for b in range(BSZ):
                    pltpu.async_copy(tabs[b].at[subc[sl]], gs[sl][b], sg)
                for b in range(BSZ):
                    pltpu.make_async_copy(tabs[b].at[subc[sl]], gs[sl][b],
                                          sg).wait()
                def mul_body(v, carry2):
                    vsl = pl.ds(v * LANES, LANES)
                    for b in range(BSZ):
                        gs[sl][b][vsl] = gs[sl][b][vsl] * pcs[sl][b][vsl]
                    return carry2
                lax.fori_loop(0, CHUNK // LANES, mul_body, 0)
                for b in range(BSZ):
                    pltpu.async_copy(gs[sl][b], accs[b].at[objc[sl]],
                                     sem_s[sl], add=True)

            @pl.when(slot == 0)
            def _():
                slot_body(0, 1)

            @pl.when(slot == 1)
            def _():
                slot_body(1, 0)
            return carry
        lax.fori_loop(0, nchunk, chunk_body, 0)

        last = nchunk - 1
        wait_scatters(last % 2)
        wait_inputs(nchunk % 2)

        plsc.subcore_barrier()
        for b in range(BSZ):
            pltpu.sync_copy(accs[b].at[pl.ds(ent0, rpt)], stage)
            pltpu.sync_copy(
                stage,
                out_hbm.at[pl.ds((c * BSZ + b) * nent_pad + ent0, rpt)])

    return step(tab_in, sub, obj, p, zeros)


def kernel(e_s, pair, d_prob, hop_attn_logits, q_mask_logits):
    num_steps, bsz, E = d_prob.shape
    num_ent = e_s.shape[1]

    nent_pad = -(-num_ent // (NS * LANES)) * (NS * LANES)
    e_pad = -(-E // (NW * CHUNK)) * (NW * CHUNK)
    ept = e_pad // NW

    sub = pair[:, 0]
    obj = pair[:, 1]
    if e_pad != E:
        sub = jnp.pad(sub, (0, e_pad - E))
        obj = jnp.pad(obj, (0, e_pad - E))
        d_prob = jnp.pad(d_prob, ((0, 0), (0, 0), (0, e_pad - E)))

    zeros = jnp.zeros((nent_pad,), jnp.float32)
    tab = jnp.pad(e_s, ((0, 0), (0, nent_pad - num_ent)))

    ent_probs = []
    for t in range(num_steps):
        parts = _step_call(tab.reshape(-1), sub, obj,
                           d_prob[t].reshape(-1), zeros, nent_pad, ept)
        parts = parts.reshape(NC, bsz, nent_pad)
        new_e = parts[0] + parts[1]
        tab = new_e / jnp.maximum(new_e, 1.0)
        ent_probs.append(tab[:, :num_ent])

    hop_attn = jax.nn.softmax(hop_attn_logits, axis=1)
    last_e = sum(ent_probs[t] * hop_attn[:, t:t + 1] for t in range(num_steps))
    m = (jnp.argmax(hop_attn, axis=1) == 1).astype(jnp.float32)[:, None] * e_s
    last_e = (1.0 - m) * last_e
    last_e = last_e * jax.nn.sigmoid(q_mask_logits)
    return last_e
